# baseline (speedup 1.0000x reference)
import functools

import jax
import jax.numpy as jnp
from jax import lax
from jax.experimental import pallas as pl
from jax.experimental.pallas import tpu as pltpu
from jax.experimental.pallas import tpu_sc as plsc

N_FLAT = 20000
N_TOT = 10000
E = 160000
D = 256
DE = 16

NS = 16
NC = 2
E_PAD = 163840
EPT = E_PAD // NS
NBLK = EPT // 128

NF_PAD = 20480
NT_PAD = 10240

_MESH = plsc.VectorSubcoreMesh(core_axis_name="c", subcore_axis_name="s")
_SC_PARAMS = pltpu.CompilerParams(use_tc_tiling_on_sc=False)


def _proj_body(ea_ref, we_ref, be_ref, *outs):
    res = (jnp.dot(ea_ref[...], we_ref[...],
                   preferred_element_type=jnp.float32) + be_ref[...])
    for i, o in enumerate(outs):
        o[...] = res[:, i * 64:(i + 1) * 64]


def _edge_proj(ea_pad, We, be):
    ep = ea_pad.shape[0]
    r = 2048
    return pl.pallas_call(
        _proj_body,
        grid=(ep // r,),
        in_specs=[
            pl.BlockSpec((r, DE), lambda i: (i, 0)),
            pl.BlockSpec((DE, D), lambda i: (0, 0)),
            pl.BlockSpec((1, D), lambda i: (0, 0)),
        ],
        out_specs=[pl.BlockSpec((r, 64), lambda i: (i, 0))] * 4,
        out_shape=[jax.ShapeDtypeStruct((ep, 64), jnp.float32)] * 4,
    )(ea_pad, We, be.reshape(1, D))


def _make_mlp_body(nx, nagg):
    def body(*refs):
        eps_ref = refs[0]
        xs = refs[1:1 + nx]
        aggs = refs[1 + nx:1 + nx + nagg]
        w1_ref, b1_ref, w2_ref, b2_ref = refs[1 + nx + nagg:5 + nx + nagg]
        y_ref, stats_ref, acc_ref = refs[5 + nx + nagg:]
        i = pl.program_id(0)

        @pl.when(i == 0)
        def _():
            acc_ref[...] = jnp.zeros_like(acc_ref)

        xin = jnp.concatenate([x[...] for x in xs], axis=1)
        agg = jnp.concatenate([a[...] for a in aggs], axis=1)
        xin = (1.0 + eps_ref[0]) * xin + agg
        t = jnp.maximum(
            jnp.dot(xin, w1_ref[...], preferred_element_type=jnp.float32)
            + b1_ref[...], 0.0)
        y = (jnp.dot(t, w2_ref[...], preferred_element_type=jnp.float32)
             + b2_ref[...])
        y_ref[...] = y
        acc_ref[0:1, :] += jnp.sum(y, axis=0, keepdims=True)
        acc_ref[1:2, :] += jnp.sum(y * y, axis=0, keepdims=True)

        @pl.when(i == pl.num_programs(0) - 1)
        def _():
            stats_ref[...] = acc_ref[...]
    return body


def _mlp(eps, xs, x_specs, aggs, W1, b1, W2, b2, n):
    r = 1000
    chunk = aggs[0].shape[1]
    agg_spec = pl.BlockSpec((r, chunk), lambda i: (i, 0))
    return pl.pallas_call(
        _make_mlp_body(len(xs), len(aggs)),
        grid=(n // r,),
        in_specs=[pl.BlockSpec(memory_space=pltpu.SMEM)] + list(x_specs)
        + [agg_spec] * len(aggs) + [
            pl.BlockSpec((D, D), lambda i: (0, 0)),
            pl.BlockSpec((1, D), lambda i: (0, 0)),
            pl.BlockSpec((D, D), lambda i: (0, 0)),
            pl.BlockSpec((1, D), lambda i: (0, 0)),
        ],
        out_specs=[
            pl.BlockSpec((r, D), lambda i: (i, 0)),
            pl.BlockSpec((8, D), lambda i: (0, 0)),
        ],
        out_shape=[
            jax.ShapeDtypeStruct((n, D), jnp.float32),
            jax.ShapeDtypeStruct((8, D), jnp.float32),
        ],
        scratch_shapes=[pltpu.VMEM((8, D), jnp.float32)],
    )(eps.reshape(1), *xs, *aggs, W1, b1.reshape(1, D), W2, b2.reshape(1, D))


XRT = NT_PAD // NS
XFB = XRT // 128
XB = NF_PAD // NS // 128


@functools.partial(
    pl.kernel,
    out_type=[jax.ShapeDtypeStruct((NT_PAD, 64), jnp.float32)] * 4,
    mesh=_MESH,
    compiler_params=_SC_PARAMS,
    scratch_types=[
        pltpu.VMEM_SHARED((NT_PAD, 128), jnp.float32),
        pltpu.VMEM_SHARED((NT_PAD, 16), jnp.float32),
        pltpu.VMEM((XB, 128), jnp.int32),
        pltpu.VMEM((128, 128), jnp.float32),
        pltpu.VMEM((128, 16), jnp.float32),
    ],
)
def _xsum_kernel(h_hbm, nid_hbm, zeros_hbm, zcnt_hbm, ones_hbm,
                 x0_hbm, x1_hbm, x2_hbm, x3_hbm,
                 accum, cntm, nid_v, hbuf, ones_v):
    c = lax.axis_index("c")
    s = lax.axis_index("s")
    pltpu.sync_copy(zeros_hbm, accum.at[pl.ds(s * XRT, XRT)])
    pltpu.sync_copy(zcnt_hbm, cntm.at[pl.ds(s * XRT, XRT)])
    pltpu.sync_copy(ones_hbm, ones_v)
    for b in range(XB):
        pltpu.sync_copy(nid_hbm.at[pl.ds((s * XB + b) * 128, 128)],
                        nid_v.at[b])
    plsc.subcore_barrier()

    def scatter_phase(col0):
        def blk(b, carry):
            r0 = (s * XB + b) * 128
            pltpu.sync_copy(h_hbm.at[pl.ds(r0, 128), pl.ds(col0, 128)], hbuf)
            pltpu.sync_copy(hbuf, accum.at[nid_v.at[b]], add=True)
            pltpu.sync_copy(ones_v, cntm.at[nid_v.at[b]], add=True)
            return carry
        lax.fori_loop(0, XB, blk, 0)

    pl.when(c == 0)(lambda: scatter_phase(0))
    pl.when(c == 1)(lambda: scatter_phase(128))
    plsc.subcore_barrier()

    def finalize(lo_hbm, hi_hbm):
        def fblk(b, carry):
            r0 = s * XRT + b * 128
            pltpu.sync_copy(accum.at[pl.ds(r0, 128)], hbuf)
            pltpu.sync_copy(cntm.at[pl.ds(r0, 128)], ones_v)

            def fin(r, cc):
                rec = 1.0 / jnp.maximum(ones_v[r], 1.0)
                for k in range(8):
                    sl = pl.ds(k * 16, 16)
                    hbuf[r, sl] = hbuf[r, sl] * rec
                return cc
            lax.fori_loop(0, 128, fin, 0)
            pltpu.sync_copy(hbuf.at[:, pl.ds(0, 64)],
                            lo_hbm.at[pl.ds(r0, 128)])
            pltpu.sync_copy(hbuf.at[:, pl.ds(64, 64)],
                            hi_hbm.at[pl.ds(r0, 128)])
            return carry
        lax.fori_loop(0, XFB, fblk, 0)

    pl.when(c == 0)(lambda: finalize(x0_hbm, x1_hbm))
    pl.when(c == 1)(lambda: finalize(x2_hbm, x3_hbm))


def _make_agg(n_rows, n_rows_pad):
    rpt = n_rows_pad // NS
    last = NS - 1
    cnt_last = n_rows - last * rpt

    scratch = [
        pltpu.VMEM_SHARED((n_rows_pad, 64), jnp.float32),
        pltpu.VMEM((NBLK, 128), jnp.int32),
        pltpu.VMEM((NBLK, 128), jnp.int32),
        pltpu.VMEM((128, 64), jnp.float32),
        pltpu.VMEM((128, 64), jnp.float32),
        pltpu.SemaphoreType.DMA,
    ]

    def body(t0, t1, t2, t3, src_hbm, dst_hbm, p0, p1, p2, p3, zeros_hbm,
             o0, o1, o2, o3, accum, src_v, dst_v, gbuf, pbuf, sem):
        tabs = (t0, t1, t2, t3)
        projs = (p0, p1, p2, p3)
        outs = (o0, o1, o2, o3)
        c = lax.axis_index("c")
        s = lax.axis_index("s")
        pltpu.sync_copy(src_hbm.at[pl.ds(s * NBLK, NBLK)], src_v)
        pltpu.sync_copy(dst_hbm.at[pl.ds(s * NBLK, NBLK)], dst_v)

        def run_passes(ci):
            for p in range(2):
                t = 2 * ci + p
                tab, proj, out_hbm = tabs[t], projs[t], outs[t]
                pltpu.sync_copy(zeros_hbm, accum.at[pl.ds(s * rpt, rpt)])
                plsc.subcore_barrier()

                def blk(j, carry):
                    e0 = s * EPT + j * 128
                    pltpu.async_copy(tab.at[src_v.at[j]], gbuf, sem).wait()
                    pltpu.sync_copy(proj.at[pl.ds(e0, 128)], pbuf)

                    def row(r, cc):
                        for k in range(4):
                            sl = pl.ds(k * 16, 16)
                            gbuf[r, sl] = jnp.maximum(
                                gbuf[r, sl] + pbuf[r, sl], 0.0)
                        return cc
                    lax.fori_loop(0, 128, row, 0)
                    pltpu.sync_copy(gbuf, accum.at[dst_v.at[j]], add=True)
                    return carry
                lax.fori_loop(0, NBLK, blk, 0)
                plsc.subcore_barrier()

                pl.when(s != last)(lambda: pltpu.sync_copy(
                    accum.at[pl.ds(s * rpt, rpt)],
                    out_hbm.at[pl.ds(s * rpt, rpt)]))
                pl.when(s == last)(lambda: pltpu.sync_copy(
                    accum.at[pl.ds(last * rpt, cnt_last)],
                    out_hbm.at[pl.ds(last * rpt, cnt_last)]))
                plsc.subcore_barrier()

        pl.when(c == 0)(lambda: run_passes(0))
        pl.when(c == 1)(lambda: run_passes(1))

    return functools.partial(
        pl.kernel, body,
        out_type=[jax.ShapeDtypeStruct((n_rows, 64), jnp.float32)] * 4,
        mesh=_MESH,
        compiler_params=_SC_PARAMS,
        scratch_types=scratch,
    )()


_agg1 = _make_agg(N_FLAT, NF_PAD)
_agg2 = _make_agg(N_TOT, NT_PAD)


NFB = 156
NTAIL = 32


@functools.partial(
    pl.kernel,
    out_type=jax.ShapeDtypeStruct((N_FLAT, D), jnp.float32),
    mesh=_MESH,
    compiler_params=_SC_PARAMS,
    scratch_types=[
        pltpu.VMEM((128,), jnp.int32),
        pltpu.VMEM((32,), jnp.int32),
        pltpu.VMEM((128, D), jnp.float32),
        pltpu.VMEM((128, D), jnp.float32),
        pltpu.VMEM((4, D), jnp.float32),
        pltpu.SemaphoreType.DMA,
    ],
)
def _final_kernel(h1_hbm, h2_hbm, nid_hbm, aff_hbm, out_hbm,
                  nid_v, nid_t, b1v, b2v, aff_v, sem):
    c = lax.axis_index("c")
    s = lax.axis_index("s")
    w = s * NC + c
    pltpu.sync_copy(aff_hbm, aff_v)

    def compute_rows(nrows, h1buf, h2buf):
        def row(r, cc):
            for k in range(16):
                sl = pl.ds(k * 16, 16)
                v = (h1buf[r, sl] * aff_v[0, sl] + aff_v[1, sl]
                     + h2buf[r, sl] * aff_v[2, sl] + aff_v[3, sl])
                h1buf[r, sl] = jnp.maximum(v, 0.0)
            return cc
        lax.fori_loop(0, nrows, row, 0)

    def blk(t, carry):
        bi = w + 32 * t

        @pl.when(bi < NFB)
        def _():
            r0 = bi * 128
            pltpu.sync_copy(nid_hbm.at[pl.ds(r0, 128)], nid_v)
            pltpu.sync_copy(h1_hbm.at[pl.ds(r0, 128)], b1v)
            pltpu.async_copy(h2_hbm.at[nid_v], b2v, sem).wait()
            compute_rows(128, b1v, b2v)
            pltpu.sync_copy(b1v, out_hbm.at[pl.ds(r0, 128)])
        return carry
    lax.fori_loop(0, 5, blk, 0)

    @pl.when(w == 31)
    def _():
        r0 = NFB * 128
        pltpu.sync_copy(nid_hbm.at[pl.ds(r0, NTAIL)], nid_t)
        pltpu.sync_copy(h1_hbm.at[pl.ds(r0, NTAIL)], b1v.at[pl.ds(0, NTAIL)])
        pltpu.async_copy(h2_hbm.at[nid_t], b2v.at[pl.ds(0, NTAIL)],
                         sem).wait()
        compute_rows(NTAIL, b1v, b2v)
        pltpu.sync_copy(b1v.at[pl.ds(0, NTAIL)], out_hbm.at[pl.ds(r0, NTAIL)])


def _bn_affine(stats, gamma, beta, n):
    mu = stats[0] / n
    var = stats[1] / n - mu * mu
    sc = gamma * jax.lax.rsqrt(var + 1e-5)
    return sc, beta - mu * sc


def kernel(h_flat, intra_ei, ea_flat, valid, node_ids, N_total, edge_index,
           edge_attr, eps_l, We_l, be_l, W1_l, b1_l, W2_l, b2_l, g_l, bt_l,
           eps_g, We_g, be_g, W1_g, b1_g, W2_g, b2_g, g_g, bt_g):
    f32 = jnp.float32
    pad_e = E_PAD - E

    src1 = jnp.pad(intra_ei[0], (0, pad_e)).reshape(E_PAD // 128, 128)
    dst1 = jnp.pad(intra_ei[1], (0, pad_e),
                   constant_values=N_FLAT).reshape(E_PAD // 128, 128)
    src2 = jnp.pad(edge_index[0], (0, pad_e)).reshape(E_PAD // 128, 128)
    dst2 = jnp.pad(edge_index[1], (0, pad_e),
                   constant_values=N_TOT).reshape(E_PAD // 128, 128)
    nid_pad = jnp.pad(node_ids, (0, NF_PAD - N_FLAT), constant_values=N_TOT)
    h_pad = jnp.pad(h_flat, ((0, NF_PAD - N_FLAT), (0, 0)))

    ea1 = jnp.pad(ea_flat, ((0, pad_e), (0, 0)))
    ea2 = jnp.pad(edge_attr, ((0, pad_e), (0, 0)))
    proj1 = _edge_proj(ea1, We_l, be_l)
    proj2 = _edge_proj(ea2, We_g, be_g)

    zeros1 = jnp.zeros((NF_PAD // NS, 64), f32)
    zeros2 = jnp.zeros((NT_PAD // NS, 64), f32)
    zeros2w = jnp.zeros((NT_PAD // NS, 128), f32)
    zcnt = jnp.zeros((NT_PAD // NS, 16), f32)
    ones_c = jnp.ones((128, 16), f32)

    xt = _xsum_kernel(h_pad, nid_pad, zeros2w, zcnt, ones_c)

    t1 = [h_flat[:, 0:64], h_flat[:, 64:128],
          h_flat[:, 128:192], h_flat[:, 192:256]]
    agg1 = _agg1(*t1, src1, dst1, *proj1, zeros1)
    two_col = [pl.BlockSpec((1000, 128), lambda i: (i, 0)),
               pl.BlockSpec((1000, 128), lambda i: (i, 1))]
    h1, stats1 = _mlp(eps_l, [h_flat, h_flat], two_col, agg1,
                      W1_l, b1_l, W2_l, b2_l, N_FLAT)

    agg2 = _agg2(*xt, src2, dst2, *proj2, zeros2)
    x_specs = [pl.BlockSpec((1000, 64), lambda i: (i, 0))] * 4
    h2, stats2 = _mlp(eps_g, list(xt), x_specs, agg2,
                      W1_g, b1_g, W2_g, b2_g, N_TOT)

    s_l, t_l = _bn_affine(stats1, g_l, bt_l, float(N_FLAT))
    s_g, t_g = _bn_affine(stats2, g_g, bt_g, float(N_TOT))
    aff = jnp.stack([s_l, t_l, s_g, t_g])

    return _final_kernel(h1, h2, node_ids, aff)


# speedup vs baseline: 1.1952x; 1.1952x over previous
import functools

import jax
import jax.numpy as jnp
from jax import lax
from jax.experimental import pallas as pl
from jax.experimental.pallas import tpu as pltpu
from jax.experimental.pallas import tpu_sc as plsc

N_FLAT = 20000
N_TOT = 10000
E = 160000
D = 256
DE = 16

NS = 16
NC = 2
E_PAD = 163840
EPT = E_PAD // NS
NBLK = EPT // 128

NF_PAD = 20480
NT_PAD = 10240

_MESH = plsc.VectorSubcoreMesh(core_axis_name="c", subcore_axis_name="s")
_SC_PARAMS = pltpu.CompilerParams(use_tc_tiling_on_sc=False)


def _proj_body(ea_ref, we_ref, be_ref, *outs):
    res = (jnp.dot(ea_ref[...], we_ref[...],
                   preferred_element_type=jnp.float32) + be_ref[...])
    for i, o in enumerate(outs):
        o[...] = res[:, i * 64:(i + 1) * 64]


def _edge_proj(ea_pad, We, be):
    ep = ea_pad.shape[0]
    r = 2048
    return pl.pallas_call(
        _proj_body,
        grid=(ep // r,),
        in_specs=[
            pl.BlockSpec((r, DE), lambda i: (i, 0)),
            pl.BlockSpec((DE, D), lambda i: (0, 0)),
            pl.BlockSpec((1, D), lambda i: (0, 0)),
        ],
        out_specs=[pl.BlockSpec((r, 64), lambda i: (i, 0))] * 4,
        out_shape=[jax.ShapeDtypeStruct((ep, 64), jnp.float32)] * 4,
    )(ea_pad, We, be.reshape(1, D))


def _make_mlp_body(nx, nagg):
    def body(*refs):
        eps_ref = refs[0]
        xs = refs[1:1 + nx]
        aggs = refs[1 + nx:1 + nx + nagg]
        w1_ref, b1_ref, w2_ref, b2_ref = refs[1 + nx + nagg:5 + nx + nagg]
        y_ref, stats_ref, acc_ref = refs[5 + nx + nagg:]
        i = pl.program_id(0)

        @pl.when(i == 0)
        def _():
            acc_ref[...] = jnp.zeros_like(acc_ref)

        xin = jnp.concatenate([x[...] for x in xs], axis=1)
        agg = jnp.concatenate([a[...] for a in aggs], axis=1)
        xin = (1.0 + eps_ref[0]) * xin + agg
        t = jnp.maximum(
            jnp.dot(xin, w1_ref[...], preferred_element_type=jnp.float32)
            + b1_ref[...], 0.0)
        y = (jnp.dot(t, w2_ref[...], preferred_element_type=jnp.float32)
             + b2_ref[...])
        y_ref[...] = y
        acc_ref[0:1, :] += jnp.sum(y, axis=0, keepdims=True)
        acc_ref[1:2, :] += jnp.sum(y * y, axis=0, keepdims=True)

        @pl.when(i == pl.num_programs(0) - 1)
        def _():
            stats_ref[...] = acc_ref[...]
    return body


def _mlp(eps, xs, x_specs, aggs, W1, b1, W2, b2, n):
    r = 1000
    chunk = aggs[0].shape[1]
    agg_spec = pl.BlockSpec((r, chunk), lambda i: (i, 0))
    return pl.pallas_call(
        _make_mlp_body(len(xs), len(aggs)),
        grid=(n // r,),
        in_specs=[pl.BlockSpec(memory_space=pltpu.SMEM)] + list(x_specs)
        + [agg_spec] * len(aggs) + [
            pl.BlockSpec((D, D), lambda i: (0, 0)),
            pl.BlockSpec((1, D), lambda i: (0, 0)),
            pl.BlockSpec((D, D), lambda i: (0, 0)),
            pl.BlockSpec((1, D), lambda i: (0, 0)),
        ],
        out_specs=[
            pl.BlockSpec((r, D), lambda i: (i, 0)),
            pl.BlockSpec((8, D), lambda i: (0, 0)),
        ],
        out_shape=[
            jax.ShapeDtypeStruct((n, D), jnp.float32),
            jax.ShapeDtypeStruct((8, D), jnp.float32),
        ],
        scratch_shapes=[pltpu.VMEM((8, D), jnp.float32)],
    )(eps.reshape(1), *xs, *aggs, W1, b1.reshape(1, D), W2, b2.reshape(1, D))


XRT = NT_PAD // NS
XFB = XRT // 128
XB = NF_PAD // NS // 128


@functools.partial(
    pl.kernel,
    out_type=[jax.ShapeDtypeStruct((NT_PAD, 64), jnp.float32)] * 4,
    mesh=_MESH,
    compiler_params=_SC_PARAMS,
    scratch_types=[
        pltpu.VMEM_SHARED((NT_PAD, 128), jnp.float32),
        pltpu.VMEM_SHARED((NT_PAD, 16), jnp.float32),
        pltpu.VMEM((XB, 128), jnp.int32),
        pltpu.VMEM((128, 128), jnp.float32),
        pltpu.VMEM((128, 16), jnp.float32),
    ],
)
def _xsum_kernel(h_hbm, nid_hbm, zeros_hbm, zcnt_hbm, ones_hbm,
                 x0_hbm, x1_hbm, x2_hbm, x3_hbm,
                 accum, cntm, nid_v, hbuf, ones_v):
    c = lax.axis_index("c")
    s = lax.axis_index("s")
    pltpu.sync_copy(zeros_hbm, accum.at[pl.ds(s * XRT, XRT)])
    pltpu.sync_copy(zcnt_hbm, cntm.at[pl.ds(s * XRT, XRT)])
    pltpu.sync_copy(ones_hbm, ones_v)
    for b in range(XB):
        pltpu.sync_copy(nid_hbm.at[pl.ds((s * XB + b) * 128, 128)],
                        nid_v.at[b])
    plsc.subcore_barrier()

    def scatter_phase(col0):
        def blk(b, carry):
            r0 = (s * XB + b) * 128
            pltpu.sync_copy(h_hbm.at[pl.ds(r0, 128), pl.ds(col0, 128)], hbuf)
            pltpu.sync_copy(hbuf, accum.at[nid_v.at[b]], add=True)
            pltpu.sync_copy(ones_v, cntm.at[nid_v.at[b]], add=True)
            return carry
        lax.fori_loop(0, XB, blk, 0)

    pl.when(c == 0)(lambda: scatter_phase(0))
    pl.when(c == 1)(lambda: scatter_phase(128))
    plsc.subcore_barrier()

    def finalize(lo_hbm, hi_hbm):
        def fblk(b, carry):
            r0 = s * XRT + b * 128
            pltpu.sync_copy(accum.at[pl.ds(r0, 128)], hbuf)
            pltpu.sync_copy(cntm.at[pl.ds(r0, 128)], ones_v)

            def fin(r, cc):
                rec = 1.0 / jnp.maximum(ones_v[r], 1.0)
                for k in range(8):
                    sl = pl.ds(k * 16, 16)
                    hbuf[r, sl] = hbuf[r, sl] * rec
                return cc
            lax.fori_loop(0, 128, fin, 0)
            pltpu.sync_copy(hbuf.at[:, pl.ds(0, 64)],
                            lo_hbm.at[pl.ds(r0, 128)])
            pltpu.sync_copy(hbuf.at[:, pl.ds(64, 64)],
                            hi_hbm.at[pl.ds(r0, 128)])
            return carry
        lax.fori_loop(0, XFB, fblk, 0)

    pl.when(c == 0)(lambda: finalize(x0_hbm, x1_hbm))
    pl.when(c == 1)(lambda: finalize(x2_hbm, x3_hbm))


def _make_agg(n_rows, n_rows_pad):
    rpt = n_rows_pad // NS
    last = NS - 1
    cnt_last = n_rows - last * rpt
    QB = 20
    NQ = NBLK // QB

    scratch = [
        pltpu.VMEM_SHARED((n_rows_pad, 64), jnp.float32),
        pltpu.VMEM((QB, 128), jnp.int32),
        pltpu.VMEM((QB, 128), jnp.int32),
        pltpu.VMEM((128, 64), jnp.float32),
        pltpu.VMEM((128, 64), jnp.float32),
        pltpu.VMEM((128, 64), jnp.float32),
        pltpu.VMEM((128, 64), jnp.float32),
    ] + [pltpu.SemaphoreType.DMA] * 6

    def body(t0, t1, t2, t3, src_hbm, dst_hbm, p0, p1, p2, p3, zeros_hbm,
             o0, o1, o2, o3, accum, src_v, dst_v, gbufA, gbufB, pbufA, pbufB,
             gsA, gsB, psA, psB, ssA, ssB):
        tabs = (t0, t1, t2, t3)
        projs = (p0, p1, p2, p3)
        outs = (o0, o1, o2, o3)
        c = lax.axis_index("c")
        s = lax.axis_index("s")

        def compute(gb, pb):
            def rows(r4, cc):
                for dr in range(4):
                    r = r4 * 4 + dr
                    for k in range(4):
                        sl = pl.ds(k * 16, 16)
                        gb[r, sl] = jnp.maximum(gb[r, sl] + pb[r, sl], 0.0)
                return cc
            lax.fori_loop(0, 32, rows, 0)

        def run_pass(tab, proj, out_hbm):
            pltpu.sync_copy(zeros_hbm, accum.at[pl.ds(s * rpt, rpt)])
            plsc.subcore_barrier()

            def issue(lrow, q, gb, pb, gsem, psem):
                e0 = s * EPT + (q * QB + lrow) * 128
                pltpu.async_copy(tab.at[src_v.at[lrow]], gb, gsem)
                pltpu.async_copy(proj.at[pl.ds(e0, 128)], pb, psem)

            def wait_gp(lrow, q, gb, pb, gsem, psem):
                e0 = s * EPT + (q * QB + lrow) * 128
                pltpu.make_async_copy(tab.at[src_v.at[lrow]], gb, gsem).wait()
                pltpu.make_async_copy(proj.at[pl.ds(e0, 128)], pb,
                                      psem).wait()

            def quarter(q, cc):
                @pl.when(q > 0)
                def _():
                    pltpu.make_async_copy(
                        gbufA, accum.at[dst_v.at[QB - 2]], ssA).wait()
                    pltpu.make_async_copy(
                        gbufB, accum.at[dst_v.at[QB - 1]], ssB).wait()
                r0 = s * NBLK + q * QB
                pltpu.sync_copy(src_hbm.at[pl.ds(r0, QB)], src_v)
                pltpu.sync_copy(dst_hbm.at[pl.ds(r0, QB)], dst_v)
                issue(0, q, gbufA, pbufA, gsA, psA)
                issue(1, q, gbufB, pbufB, gsB, psB)

                def pair(t, tc):
                    l0 = 2 * t
                    l1 = l0 + 1
                    wait_gp(l0, q, gbufA, pbufA, gsA, psA)
                    compute(gbufA, pbufA)
                    pltpu.async_copy(gbufA, accum.at[dst_v.at[l0]], ssA,
                                     add=True)
                    wait_gp(l1, q, gbufB, pbufB, gsB, psB)
                    compute(gbufB, pbufB)

                    @pl.when(t < QB // 2 - 1)
                    def _():
                        pltpu.make_async_copy(
                            gbufA, accum.at[dst_v.at[l0]], ssA).wait()
                        issue(l0 + 2, q, gbufA, pbufA, gsA, psA)
                    pltpu.async_copy(gbufB, accum.at[dst_v.at[l1]], ssB,
                                     add=True)

                    @pl.when(t < QB // 2 - 1)
                    def _():
                        pltpu.make_async_copy(
                            gbufB, accum.at[dst_v.at[l1]], ssB).wait()
                        issue(l1 + 2, q, gbufB, pbufB, gsB, psB)
                    return tc
                lax.fori_loop(0, QB // 2, pair, 0)
                return cc
            lax.fori_loop(0, NQ, quarter, 0)
            pltpu.make_async_copy(gbufA, accum.at[dst_v.at[QB - 2]],
                                  ssA).wait()
            pltpu.make_async_copy(gbufB, accum.at[dst_v.at[QB - 1]],
                                  ssB).wait()
            plsc.subcore_barrier()

            pl.when(s != last)(lambda: pltpu.sync_copy(
                accum.at[pl.ds(s * rpt, rpt)],
                out_hbm.at[pl.ds(s * rpt, rpt)]))
            pl.when(s == last)(lambda: pltpu.sync_copy(
                accum.at[pl.ds(last * rpt, cnt_last)],
                out_hbm.at[pl.ds(last * rpt, cnt_last)]))
            plsc.subcore_barrier()

        def run_passes(ci):
            for p in range(2):
                t = 2 * ci + p
                run_pass(tabs[t], projs[t], outs[t])

        pl.when(c == 0)(lambda: run_passes(0))
        pl.when(c == 1)(lambda: run_passes(1))

    return functools.partial(
        pl.kernel, body,
        out_type=[jax.ShapeDtypeStruct((n_rows, 64), jnp.float32)] * 4,
        mesh=_MESH,
        compiler_params=_SC_PARAMS,
        scratch_types=scratch,
    )()


_agg1 = _make_agg(N_FLAT, NF_PAD)
_agg2 = _make_agg(N_TOT, NT_PAD)


NFB = 156
NTAIL = 32


@functools.partial(
    pl.kernel,
    out_type=jax.ShapeDtypeStruct((N_FLAT, D), jnp.float32),
    mesh=_MESH,
    compiler_params=_SC_PARAMS,
    scratch_types=[
        pltpu.VMEM((128,), jnp.int32),
        pltpu.VMEM((32,), jnp.int32),
        pltpu.VMEM((128, D), jnp.float32),
        pltpu.VMEM((128, D), jnp.float32),
        pltpu.VMEM((4, D), jnp.float32),
        pltpu.SemaphoreType.DMA,
    ],
)
def _final_kernel(h1_hbm, h2_hbm, nid_hbm, aff_hbm, out_hbm,
                  nid_v, nid_t, b1v, b2v, aff_v, sem):
    c = lax.axis_index("c")
    s = lax.axis_index("s")
    w = s * NC + c
    pltpu.sync_copy(aff_hbm, aff_v)

    def compute_rows(nrows, h1buf, h2buf):
        def row(r, cc):
            for k in range(16):
                sl = pl.ds(k * 16, 16)
                v = (h1buf[r, sl] * aff_v[0, sl] + aff_v[1, sl]
                     + h2buf[r, sl] * aff_v[2, sl] + aff_v[3, sl])
                h1buf[r, sl] = jnp.maximum(v, 0.0)
            return cc
        lax.fori_loop(0, nrows, row, 0)

    def blk(t, carry):
        bi = w + 32 * t

        @pl.when(bi < NFB)
        def _():
            r0 = bi * 128
            pltpu.sync_copy(nid_hbm.at[pl.ds(r0, 128)], nid_v)
            pltpu.sync_copy(h1_hbm.at[pl.ds(r0, 128)], b1v)
            pltpu.async_copy(h2_hbm.at[nid_v], b2v, sem).wait()
            compute_rows(128, b1v, b2v)
            pltpu.sync_copy(b1v, out_hbm.at[pl.ds(r0, 128)])
        return carry
    lax.fori_loop(0, 5, blk, 0)

    @pl.when(w == 31)
    def _():
        r0 = NFB * 128
        pltpu.sync_copy(nid_hbm.at[pl.ds(r0, NTAIL)], nid_t)
        pltpu.sync_copy(h1_hbm.at[pl.ds(r0, NTAIL)], b1v.at[pl.ds(0, NTAIL)])
        pltpu.async_copy(h2_hbm.at[nid_t], b2v.at[pl.ds(0, NTAIL)],
                         sem).wait()
        compute_rows(NTAIL, b1v, b2v)
        pltpu.sync_copy(b1v.at[pl.ds(0, NTAIL)], out_hbm.at[pl.ds(r0, NTAIL)])


def _bn_affine(stats, gamma, beta, n):
    mu = stats[0] / n
    var = stats[1] / n - mu * mu
    sc = gamma * jax.lax.rsqrt(var + 1e-5)
    return sc, beta - mu * sc


def kernel(h_flat, intra_ei, ea_flat, valid, node_ids, N_total, edge_index,
           edge_attr, eps_l, We_l, be_l, W1_l, b1_l, W2_l, b2_l, g_l, bt_l,
           eps_g, We_g, be_g, W1_g, b1_g, W2_g, b2_g, g_g, bt_g):
    f32 = jnp.float32
    pad_e = E_PAD - E

    src1 = jnp.pad(intra_ei[0], (0, pad_e)).reshape(E_PAD // 128, 128)
    dst1 = jnp.pad(intra_ei[1], (0, pad_e),
                   constant_values=N_FLAT).reshape(E_PAD // 128, 128)
    src2 = jnp.pad(edge_index[0], (0, pad_e)).reshape(E_PAD // 128, 128)
    dst2 = jnp.pad(edge_index[1], (0, pad_e),
                   constant_values=N_TOT).reshape(E_PAD // 128, 128)
    nid_pad = jnp.pad(node_ids, (0, NF_PAD - N_FLAT), constant_values=N_TOT)
    h_pad = jnp.pad(h_flat, ((0, NF_PAD - N_FLAT), (0, 0)))

    ea1 = jnp.pad(ea_flat, ((0, pad_e), (0, 0)))
    ea2 = jnp.pad(edge_attr, ((0, pad_e), (0, 0)))
    proj1 = _edge_proj(ea1, We_l, be_l)
    proj2 = _edge_proj(ea2, We_g, be_g)

    zeros1 = jnp.zeros((NF_PAD // NS, 64), f32)
    zeros2 = jnp.zeros((NT_PAD // NS, 64), f32)
    zeros2w = jnp.zeros((NT_PAD // NS, 128), f32)
    zcnt = jnp.zeros((NT_PAD // NS, 16), f32)
    ones_c = jnp.ones((128, 16), f32)

    xt = _xsum_kernel(h_pad, nid_pad, zeros2w, zcnt, ones_c)

    t1 = [h_flat[:, 0:64], h_flat[:, 64:128],
          h_flat[:, 128:192], h_flat[:, 192:256]]
    agg1 = _agg1(*t1, src1, dst1, *proj1, zeros1)
    two_col = [pl.BlockSpec((1000, 128), lambda i: (i, 0)),
               pl.BlockSpec((1000, 128), lambda i: (i, 1))]
    h1, stats1 = _mlp(eps_l, [h_flat, h_flat], two_col, agg1,
                      W1_l, b1_l, W2_l, b2_l, N_FLAT)

    agg2 = _agg2(*xt, src2, dst2, *proj2, zeros2)
    x_specs = [pl.BlockSpec((1000, 64), lambda i: (i, 0))] * 4
    h2, stats2 = _mlp(eps_g, list(xt), x_specs, agg2,
                      W1_g, b1_g, W2_g, b2_g, N_TOT)

    s_l, t_l = _bn_affine(stats1, g_l, bt_l, float(N_FLAT))
    s_g, t_g = _bn_affine(stats2, g_g, bt_g, float(N_TOT))
    aff = jnp.stack([s_l, t_l, s_g, t_g])

    return _final_kernel(h1, h2, node_ids, aff)


# speedup vs baseline: 1.2204x; 1.0211x over previous
import functools

import jax
import jax.numpy as jnp
from jax import lax
from jax.experimental import pallas as pl
from jax.experimental.pallas import tpu as pltpu
from jax.experimental.pallas import tpu_sc as plsc

N_FLAT = 20000
N_TOT = 10000
E = 160000
D = 256
DE = 16

NS = 16
NC = 2
E_PAD = 163840
EPT = E_PAD // NS
NBLK = EPT // 128

NF_PAD = 20480
NT_PAD = 10240

_MESH = plsc.VectorSubcoreMesh(core_axis_name="c", subcore_axis_name="s")
_SC_PARAMS = pltpu.CompilerParams(use_tc_tiling_on_sc=False)


def _make_proj_body(chunk):
    def body(ea_ref, we_ref, be_ref, *outs):
        res = (jnp.dot(ea_ref[...], we_ref[...],
                       preferred_element_type=jnp.float32) + be_ref[...])
        for i, o in enumerate(outs):
            o[...] = res[:, i * chunk:(i + 1) * chunk]
    return body


def _edge_proj(ea_pad, We, be, chunk):
    ep = ea_pad.shape[0]
    r = 2048
    nch = D // chunk
    return pl.pallas_call(
        _make_proj_body(chunk),
        grid=(ep // r,),
        in_specs=[
            pl.BlockSpec((r, DE), lambda i: (i, 0)),
            pl.BlockSpec((DE, D), lambda i: (0, 0)),
            pl.BlockSpec((1, D), lambda i: (0, 0)),
        ],
        out_specs=[pl.BlockSpec((r, chunk), lambda i: (i, 0))] * nch,
        out_shape=[jax.ShapeDtypeStruct((ep, chunk), jnp.float32)] * nch,
    )(ea_pad, We, be.reshape(1, D))


def _make_mlp_body(nx, nagg):
    def body(*refs):
        eps_ref = refs[0]
        xs = refs[1:1 + nx]
        aggs = refs[1 + nx:1 + nx + nagg]
        w1_ref, b1_ref, w2_ref, b2_ref = refs[1 + nx + nagg:5 + nx + nagg]
        y_ref, stats_ref, acc_ref = refs[5 + nx + nagg:]
        i = pl.program_id(0)

        @pl.when(i == 0)
        def _():
            acc_ref[...] = jnp.zeros_like(acc_ref)

        xin = jnp.concatenate([x[...] for x in xs], axis=1)
        agg = jnp.concatenate([a[...] for a in aggs], axis=1)
        xin = (1.0 + eps_ref[0]) * xin + agg
        t = jnp.maximum(
            jnp.dot(xin, w1_ref[...], preferred_element_type=jnp.float32)
            + b1_ref[...], 0.0)
        y = (jnp.dot(t, w2_ref[...], preferred_element_type=jnp.float32)
             + b2_ref[...])
        y_ref[...] = y
        acc_ref[0:1, :] += jnp.sum(y, axis=0, keepdims=True)
        acc_ref[1:2, :] += jnp.sum(y * y, axis=0, keepdims=True)

        @pl.when(i == pl.num_programs(0) - 1)
        def _():
            stats_ref[...] = acc_ref[...]
    return body


def _mlp(eps, xs, x_specs, aggs, W1, b1, W2, b2, n):
    r = 1000
    chunk = aggs[0].shape[1]
    agg_spec = pl.BlockSpec((r, chunk), lambda i: (i, 0))
    return pl.pallas_call(
        _make_mlp_body(len(xs), len(aggs)),
        grid=(n // r,),
        in_specs=[pl.BlockSpec(memory_space=pltpu.SMEM)] + list(x_specs)
        + [agg_spec] * len(aggs) + [
            pl.BlockSpec((D, D), lambda i: (0, 0)),
            pl.BlockSpec((1, D), lambda i: (0, 0)),
            pl.BlockSpec((D, D), lambda i: (0, 0)),
            pl.BlockSpec((1, D), lambda i: (0, 0)),
        ],
        out_specs=[
            pl.BlockSpec((r, D), lambda i: (i, 0)),
            pl.BlockSpec((8, D), lambda i: (0, 0)),
        ],
        out_shape=[
            jax.ShapeDtypeStruct((n, D), jnp.float32),
            jax.ShapeDtypeStruct((8, D), jnp.float32),
        ],
        scratch_shapes=[pltpu.VMEM((8, D), jnp.float32)],
    )(eps.reshape(1), *xs, *aggs, W1, b1.reshape(1, D), W2, b2.reshape(1, D))


XRT = NT_PAD // NS
XFB = XRT // 128
XB = NF_PAD // NS // 128


@functools.partial(
    pl.kernel,
    out_type=[jax.ShapeDtypeStruct((NT_PAD, 64), jnp.float32)] * 4,
    mesh=_MESH,
    compiler_params=_SC_PARAMS,
    scratch_types=[
        pltpu.VMEM_SHARED((NT_PAD, 128), jnp.float32),
        pltpu.VMEM_SHARED((NT_PAD, 16), jnp.float32),
        pltpu.VMEM((XB, 128), jnp.int32),
        pltpu.VMEM((128, 128), jnp.float32),
        pltpu.VMEM((128, 16), jnp.float32),
    ],
)
def _xsum_kernel(h_hbm, nid_hbm, zeros_hbm, zcnt_hbm, ones_hbm,
                 x0_hbm, x1_hbm, x2_hbm, x3_hbm,
                 accum, cntm, nid_v, hbuf, ones_v):
    c = lax.axis_index("c")
    s = lax.axis_index("s")
    pltpu.sync_copy(zeros_hbm, accum.at[pl.ds(s * XRT, XRT)])
    pltpu.sync_copy(zcnt_hbm, cntm.at[pl.ds(s * XRT, XRT)])
    pltpu.sync_copy(ones_hbm, ones_v)
    for b in range(XB):
        pltpu.sync_copy(nid_hbm.at[pl.ds((s * XB + b) * 128, 128)],
                        nid_v.at[b])
    plsc.subcore_barrier()

    def scatter_phase(col0):
        def blk(b, carry):
            r0 = (s * XB + b) * 128
            pltpu.sync_copy(h_hbm.at[pl.ds(r0, 128), pl.ds(col0, 128)], hbuf)
            pltpu.sync_copy(hbuf, accum.at[nid_v.at[b]], add=True)
            pltpu.sync_copy(ones_v, cntm.at[nid_v.at[b]], add=True)
            return carry
        lax.fori_loop(0, XB, blk, 0)

    pl.when(c == 0)(lambda: scatter_phase(0))
    pl.when(c == 1)(lambda: scatter_phase(128))
    plsc.subcore_barrier()

    def finalize(lo_hbm, hi_hbm):
        def fblk(b, carry):
            r0 = s * XRT + b * 128
            pltpu.sync_copy(accum.at[pl.ds(r0, 128)], hbuf)
            pltpu.sync_copy(cntm.at[pl.ds(r0, 128)], ones_v)

            def fin(r, cc):
                rec = 1.0 / jnp.maximum(ones_v[r], 1.0)
                for k in range(8):
                    sl = pl.ds(k * 16, 16)
                    hbuf[r, sl] = hbuf[r, sl] * rec
                return cc
            lax.fori_loop(0, 128, fin, 0)
            pltpu.sync_copy(hbuf.at[:, pl.ds(0, 64)],
                            lo_hbm.at[pl.ds(r0, 128)])
            pltpu.sync_copy(hbuf.at[:, pl.ds(64, 64)],
                            hi_hbm.at[pl.ds(r0, 128)])
            return carry
        lax.fori_loop(0, XFB, fblk, 0)

    pl.when(c == 0)(lambda: finalize(x0_hbm, x1_hbm))
    pl.when(c == 1)(lambda: finalize(x2_hbm, x3_hbm))


def _make_agg(n_rows, n_rows_pad, n_tab_rows, chunk, n_pass):
    rpt = n_rows_pad // NS
    tpt = n_tab_rows // NS
    last = NS - 1
    cnt_last = n_rows - last * rpt
    n_tab = NC * n_pass
    QB = 20
    NQ = NBLK // QB
    KG = chunk // 16

    scratch = [
        pltpu.VMEM_SHARED((n_rows_pad, chunk), jnp.float32),
        pltpu.VMEM_SHARED((n_tab_rows, chunk), jnp.float32),
        pltpu.VMEM((QB, 128), jnp.int32),
        pltpu.VMEM((QB, 128), jnp.int32),
        pltpu.VMEM((128, chunk), jnp.float32),
        pltpu.VMEM((128, chunk), jnp.float32),
        pltpu.VMEM((128, chunk), jnp.float32),
        pltpu.VMEM((128, chunk), jnp.float32),
    ] + [pltpu.SemaphoreType.DMA] * 6

    def body(*refs):
        tabs = refs[:n_tab]
        src_hbm, dst_hbm = refs[n_tab], refs[n_tab + 1]
        projs = refs[n_tab + 2:2 * n_tab + 2]
        zeros_hbm = refs[2 * n_tab + 2]
        outs = refs[2 * n_tab + 3:3 * n_tab + 3]
        (accum, tabsh, src_v, dst_v, gbufA, gbufB, pbufA, pbufB,
         gsA, gsB, psA, psB, ssA, ssB) = refs[3 * n_tab + 3:]
        c = lax.axis_index("c")
        s = lax.axis_index("s")

        def compute(gb, pb):
            def rows(r4, cc):
                for dr in range(4):
                    r = r4 * 4 + dr
                    for k in range(KG):
                        sl = pl.ds(k * 16, 16)
                        gb[r, sl] = jnp.maximum(gb[r, sl] + pb[r, sl], 0.0)
                return cc
            lax.fori_loop(0, 32, rows, 0)

        def run_pass(tab, proj, out_hbm):
            pltpu.sync_copy(zeros_hbm, accum.at[pl.ds(s * rpt, rpt)])
            pltpu.sync_copy(tab.at[pl.ds(s * tpt, tpt)],
                            tabsh.at[pl.ds(s * tpt, tpt)])
            plsc.subcore_barrier()

            def issue(lrow, q, gb, pb, gsem, psem):
                e0 = s * EPT + (q * QB + lrow) * 128
                pltpu.async_copy(tabsh.at[src_v.at[lrow]], gb, gsem)
                pltpu.async_copy(proj.at[pl.ds(e0, 128)], pb, psem)

            def wait_gp(lrow, q, gb, pb, gsem, psem):
                e0 = s * EPT + (q * QB + lrow) * 128
                pltpu.make_async_copy(tabsh.at[src_v.at[lrow]], gb,
                                      gsem).wait()
                pltpu.make_async_copy(proj.at[pl.ds(e0, 128)], pb,
                                      psem).wait()

            def quarter(q, cc):
                @pl.when(q > 0)
                def _():
                    pltpu.make_async_copy(
                        gbufA, accum.at[dst_v.at[QB - 2]], ssA).wait()
                    pltpu.make_async_copy(
                        gbufB, accum.at[dst_v.at[QB - 1]], ssB).wait()
                r0 = s * NBLK + q * QB
                pltpu.sync_copy(src_hbm.at[pl.ds(r0, QB)], src_v)
                pltpu.sync_copy(dst_hbm.at[pl.ds(r0, QB)], dst_v)
                issue(0, q, gbufA, pbufA, gsA, psA)
                issue(1, q, gbufB, pbufB, gsB, psB)

                def pair(t, tc):
                    l0 = 2 * t
                    l1 = l0 + 1
                    wait_gp(l0, q, gbufA, pbufA, gsA, psA)
                    compute(gbufA, pbufA)
                    pltpu.async_copy(gbufA, accum.at[dst_v.at[l0]], ssA,
                                     add=True)
                    wait_gp(l1, q, gbufB, pbufB, gsB, psB)
                    compute(gbufB, pbufB)

                    @pl.when(t < QB // 2 - 1)
                    def _():
                        pltpu.make_async_copy(
                            gbufA, accum.at[dst_v.at[l0]], ssA).wait()
                        issue(l0 + 2, q, gbufA, pbufA, gsA, psA)
                    pltpu.async_copy(gbufB, accum.at[dst_v.at[l1]], ssB,
                                     add=True)

                    @pl.when(t < QB // 2 - 1)
                    def _():
                        pltpu.make_async_copy(
                            gbufB, accum.at[dst_v.at[l1]], ssB).wait()
                        issue(l1 + 2, q, gbufB, pbufB, gsB, psB)
                    return tc
                lax.fori_loop(0, QB // 2, pair, 0)
                return cc
            lax.fori_loop(0, NQ, quarter, 0)
            pltpu.make_async_copy(gbufA, accum.at[dst_v.at[QB - 2]],
                                  ssA).wait()
            pltpu.make_async_copy(gbufB, accum.at[dst_v.at[QB - 1]],
                                  ssB).wait()
            plsc.subcore_barrier()

            pl.when(s != last)(lambda: pltpu.sync_copy(
                accum.at[pl.ds(s * rpt, rpt)],
                out_hbm.at[pl.ds(s * rpt, rpt)]))
            pl.when(s == last)(lambda: pltpu.sync_copy(
                accum.at[pl.ds(last * rpt, cnt_last)],
                out_hbm.at[pl.ds(last * rpt, cnt_last)]))
            plsc.subcore_barrier()

        def run_passes(ci):
            for p in range(n_pass):
                t = n_pass * ci + p
                run_pass(tabs[t], projs[t], outs[t])

        pl.when(c == 0)(lambda: run_passes(0))
        pl.when(c == 1)(lambda: run_passes(1))

    return functools.partial(
        pl.kernel, body,
        out_type=[jax.ShapeDtypeStruct((n_rows, chunk), jnp.float32)] * n_tab,
        mesh=_MESH,
        compiler_params=_SC_PARAMS,
        scratch_types=scratch,
    )()


_agg1 = _make_agg(N_FLAT, NF_PAD, N_FLAT, 32, 4)
_agg2 = _make_agg(N_TOT, NT_PAD, NT_PAD, 64, 2)


NFB = 156
NTAIL = 32


@functools.partial(
    pl.kernel,
    out_type=jax.ShapeDtypeStruct((N_FLAT, D), jnp.float32),
    mesh=_MESH,
    compiler_params=_SC_PARAMS,
    scratch_types=[
        pltpu.VMEM((128,), jnp.int32),
        pltpu.VMEM((32,), jnp.int32),
        pltpu.VMEM((128, D), jnp.float32),
        pltpu.VMEM((128, D), jnp.float32),
        pltpu.VMEM((4, D), jnp.float32),
        pltpu.SemaphoreType.DMA,
    ],
)
def _final_kernel(h1_hbm, h2_hbm, nid_hbm, aff_hbm, out_hbm,
                  nid_v, nid_t, b1v, b2v, aff_v, sem):
    c = lax.axis_index("c")
    s = lax.axis_index("s")
    w = s * NC + c
    pltpu.sync_copy(aff_hbm, aff_v)

    def compute_rows(nrows, h1buf, h2buf):
        def row(r, cc):
            for k in range(16):
                sl = pl.ds(k * 16, 16)
                v = (h1buf[r, sl] * aff_v[0, sl] + aff_v[1, sl]
                     + h2buf[r, sl] * aff_v[2, sl] + aff_v[3, sl])
                h1buf[r, sl] = jnp.maximum(v, 0.0)
            return cc
        lax.fori_loop(0, nrows, row, 0)

    def blk(t, carry):
        bi = w + 32 * t

        @pl.when(bi < NFB)
        def _():
            r0 = bi * 128
            pltpu.sync_copy(nid_hbm.at[pl.ds(r0, 128)], nid_v)
            pltpu.sync_copy(h1_hbm.at[pl.ds(r0, 128)], b1v)
            pltpu.async_copy(h2_hbm.at[nid_v], b2v, sem).wait()
            compute_rows(128, b1v, b2v)
            pltpu.sync_copy(b1v, out_hbm.at[pl.ds(r0, 128)])
        return carry
    lax.fori_loop(0, 5, blk, 0)

    @pl.when(w == 31)
    def _():
        r0 = NFB * 128
        pltpu.sync_copy(nid_hbm.at[pl.ds(r0, NTAIL)], nid_t)
        pltpu.sync_copy(h1_hbm.at[pl.ds(r0, NTAIL)], b1v.at[pl.ds(0, NTAIL)])
        pltpu.async_copy(h2_hbm.at[nid_t], b2v.at[pl.ds(0, NTAIL)],
                         sem).wait()
        compute_rows(NTAIL, b1v, b2v)
        pltpu.sync_copy(b1v.at[pl.ds(0, NTAIL)], out_hbm.at[pl.ds(r0, NTAIL)])


def _bn_affine(stats, gamma, beta, n):
    mu = stats[0] / n
    var = stats[1] / n - mu * mu
    sc = gamma * jax.lax.rsqrt(var + 1e-5)
    return sc, beta - mu * sc


def kernel(h_flat, intra_ei, ea_flat, valid, node_ids, N_total, edge_index,
           edge_attr, eps_l, We_l, be_l, W1_l, b1_l, W2_l, b2_l, g_l, bt_l,
           eps_g, We_g, be_g, W1_g, b1_g, W2_g, b2_g, g_g, bt_g):
    f32 = jnp.float32
    pad_e = E_PAD - E

    src1 = jnp.pad(intra_ei[0], (0, pad_e)).reshape(E_PAD // 128, 128)
    dst1 = jnp.pad(intra_ei[1], (0, pad_e),
                   constant_values=N_FLAT).reshape(E_PAD // 128, 128)
    src2 = jnp.pad(edge_index[0], (0, pad_e)).reshape(E_PAD // 128, 128)
    dst2 = jnp.pad(edge_index[1], (0, pad_e),
                   constant_values=N_TOT).reshape(E_PAD // 128, 128)
    nid_pad = jnp.pad(node_ids, (0, NF_PAD - N_FLAT), constant_values=N_TOT)
    h_pad = jnp.pad(h_flat, ((0, NF_PAD - N_FLAT), (0, 0)))

    ea1 = jnp.pad(ea_flat, ((0, pad_e), (0, 0)))
    ea2 = jnp.pad(edge_attr, ((0, pad_e), (0, 0)))
    proj1 = _edge_proj(ea1, We_l, be_l, 32)
    proj2 = _edge_proj(ea2, We_g, be_g, 64)

    zeros1 = jnp.zeros((NF_PAD // NS, 32), f32)
    zeros2 = jnp.zeros((NT_PAD // NS, 64), f32)
    zeros2w = jnp.zeros((NT_PAD // NS, 128), f32)
    zcnt = jnp.zeros((NT_PAD // NS, 16), f32)
    ones_c = jnp.ones((128, 16), f32)

    xt = _xsum_kernel(h_pad, nid_pad, zeros2w, zcnt, ones_c)

    t1 = [h_flat[:, i * 32:(i + 1) * 32] for i in range(8)]
    agg1 = _agg1(*t1, src1, dst1, *proj1, zeros1)
    two_col = [pl.BlockSpec((1000, 128), lambda i: (i, 0)),
               pl.BlockSpec((1000, 128), lambda i: (i, 1))]
    h1, stats1 = _mlp(eps_l, [h_flat, h_flat], two_col, agg1,
                      W1_l, b1_l, W2_l, b2_l, N_FLAT)

    agg2 = _agg2(*xt, src2, dst2, *proj2, zeros2)
    x_specs = [pl.BlockSpec((1000, 64), lambda i: (i, 0))] * 4
    h2, stats2 = _mlp(eps_g, list(xt), x_specs, agg2,
                      W1_g, b1_g, W2_g, b2_g, N_TOT)

    s_l, t_l = _bn_affine(stats1, g_l, bt_l, float(N_FLAT))
    s_g, t_g = _bn_affine(stats2, g_g, bt_g, float(N_TOT))
    aff = jnp.stack([s_l, t_l, s_g, t_g])

    return _final_kernel(h1, h2, node_ids, aff)


# speedup vs baseline: 1.2809x; 1.0496x over previous
import functools

import jax
import jax.numpy as jnp
from jax import lax
from jax.experimental import pallas as pl
from jax.experimental.pallas import tpu as pltpu
from jax.experimental.pallas import tpu_sc as plsc

N_FLAT = 20000
N_TOT = 10000
E = 160000
D = 256
DE = 16

NS = 16
NC = 2
E_PAD = 163840
EPT = E_PAD // NS
NBLK = EPT // 128

NF_PAD = 20480
NT_PAD = 10240

_MESH = plsc.VectorSubcoreMesh(core_axis_name="c", subcore_axis_name="s")
_SC_PARAMS = pltpu.CompilerParams(use_tc_tiling_on_sc=False)


def _make_proj_body(chunk):
    def body(ea_ref, we_ref, be_ref, *outs):
        res = (jnp.dot(ea_ref[...], we_ref[...],
                       preferred_element_type=jnp.float32) + be_ref[...])
        for i, o in enumerate(outs):
            o[...] = res[:, i * chunk:(i + 1) * chunk]
    return body


def _edge_proj(ea_pad, We, be, chunk):
    ep = ea_pad.shape[0]
    r = 2048
    nch = D // chunk
    return pl.pallas_call(
        _make_proj_body(chunk),
        grid=(ep // r,),
        in_specs=[
            pl.BlockSpec((r, DE), lambda i: (i, 0)),
            pl.BlockSpec((DE, D), lambda i: (0, 0)),
            pl.BlockSpec((1, D), lambda i: (0, 0)),
        ],
        out_specs=[pl.BlockSpec((r, chunk), lambda i: (i, 0))] * nch,
        out_shape=[jax.ShapeDtypeStruct((ep, chunk), jnp.float32)] * nch,
    )(ea_pad, We, be.reshape(1, D))


def _make_mlp_body(nx, nagg):
    def body(*refs):
        eps_ref = refs[0]
        xs = refs[1:1 + nx]
        aggs = refs[1 + nx:1 + nx + nagg]
        w1_ref, b1_ref, w2_ref, b2_ref = refs[1 + nx + nagg:5 + nx + nagg]
        y_ref, stats_ref, acc_ref = refs[5 + nx + nagg:]
        i = pl.program_id(0)

        @pl.when(i == 0)
        def _():
            acc_ref[...] = jnp.zeros_like(acc_ref)

        xin = jnp.concatenate([x[...] for x in xs], axis=1)
        agg = jnp.concatenate([a[...] for a in aggs], axis=1)
        xin = (1.0 + eps_ref[0]) * xin + agg
        t = jnp.maximum(
            jnp.dot(xin, w1_ref[...], preferred_element_type=jnp.float32)
            + b1_ref[...], 0.0)
        y = (jnp.dot(t, w2_ref[...], preferred_element_type=jnp.float32)
             + b2_ref[...])
        y_ref[...] = y
        acc_ref[0:1, :] += jnp.sum(y, axis=0, keepdims=True)
        acc_ref[1:2, :] += jnp.sum(y * y, axis=0, keepdims=True)

        @pl.when(i == pl.num_programs(0) - 1)
        def _():
            stats_ref[...] = acc_ref[...]
    return body


def _mlp(eps, xs, x_specs, aggs, W1, b1, W2, b2, n):
    r = 1000
    chunk = aggs[0].shape[1]
    agg_spec = pl.BlockSpec((r, chunk), lambda i: (i, 0))
    return pl.pallas_call(
        _make_mlp_body(len(xs), len(aggs)),
        grid=(n // r,),
        in_specs=[pl.BlockSpec(memory_space=pltpu.SMEM)] + list(x_specs)
        + [agg_spec] * len(aggs) + [
            pl.BlockSpec((D, D), lambda i: (0, 0)),
            pl.BlockSpec((1, D), lambda i: (0, 0)),
            pl.BlockSpec((D, D), lambda i: (0, 0)),
            pl.BlockSpec((1, D), lambda i: (0, 0)),
        ],
        out_specs=[
            pl.BlockSpec((r, D), lambda i: (i, 0)),
            pl.BlockSpec((8, D), lambda i: (0, 0)),
        ],
        out_shape=[
            jax.ShapeDtypeStruct((n, D), jnp.float32),
            jax.ShapeDtypeStruct((8, D), jnp.float32),
        ],
        scratch_shapes=[pltpu.VMEM((8, D), jnp.float32)],
    )(eps.reshape(1), *xs, *aggs, W1, b1.reshape(1, D), W2, b2.reshape(1, D))


XRT = NT_PAD // NS
XFB = XRT // 128
XB = NF_PAD // NS // 128


@functools.partial(
    pl.kernel,
    out_type=[jax.ShapeDtypeStruct((NT_PAD, 64), jnp.float32)] * 4,
    mesh=_MESH,
    compiler_params=_SC_PARAMS,
    scratch_types=[
        pltpu.VMEM_SHARED((NT_PAD, 128), jnp.float32),
        pltpu.VMEM_SHARED((NT_PAD, 16), jnp.float32),
        pltpu.VMEM((XB, 128), jnp.int32),
        pltpu.VMEM((128, 128), jnp.float32),
        pltpu.VMEM((128, 16), jnp.float32),
    ],
)
def _xsum_kernel(h_hbm, nid_hbm, zeros_hbm, zcnt_hbm, ones_hbm,
                 x0_hbm, x1_hbm, x2_hbm, x3_hbm,
                 accum, cntm, nid_v, hbuf, ones_v):
    c = lax.axis_index("c")
    s = lax.axis_index("s")
    pltpu.sync_copy(zeros_hbm, accum.at[pl.ds(s * XRT, XRT)])
    pltpu.sync_copy(zcnt_hbm, cntm.at[pl.ds(s * XRT, XRT)])
    pltpu.sync_copy(ones_hbm, ones_v)
    for b in range(XB):
        pltpu.sync_copy(nid_hbm.at[pl.ds((s * XB + b) * 128, 128)],
                        nid_v.at[b])
    plsc.subcore_barrier()

    def scatter_phase(col0):
        def blk(b, carry):
            r0 = (s * XB + b) * 128
            pltpu.sync_copy(h_hbm.at[pl.ds(r0, 128), pl.ds(col0, 128)], hbuf)
            pltpu.sync_copy(hbuf, accum.at[nid_v.at[b]], add=True)
            pltpu.sync_copy(ones_v, cntm.at[nid_v.at[b]], add=True)
            return carry
        lax.fori_loop(0, XB, blk, 0)

    pl.when(c == 0)(lambda: scatter_phase(0))
    pl.when(c == 1)(lambda: scatter_phase(128))
    plsc.subcore_barrier()

    def finalize(lo_hbm, hi_hbm):
        def fblk(b, carry):
            r0 = s * XRT + b * 128
            pltpu.sync_copy(accum.at[pl.ds(r0, 128)], hbuf)
            pltpu.sync_copy(cntm.at[pl.ds(r0, 128)], ones_v)

            def fin(r, cc):
                rec = 1.0 / jnp.maximum(ones_v[r], 1.0)
                for k in range(8):
                    sl = pl.ds(k * 16, 16)
                    hbuf[r, sl] = hbuf[r, sl] * rec
                return cc
            lax.fori_loop(0, 128, fin, 0)
            pltpu.sync_copy(hbuf.at[:, pl.ds(0, 64)],
                            lo_hbm.at[pl.ds(r0, 128)])
            pltpu.sync_copy(hbuf.at[:, pl.ds(64, 64)],
                            hi_hbm.at[pl.ds(r0, 128)])
            return carry
        lax.fori_loop(0, XFB, fblk, 0)

    pl.when(c == 0)(lambda: finalize(x0_hbm, x1_hbm))
    pl.when(c == 1)(lambda: finalize(x2_hbm, x3_hbm))


def _make_agg(n_rows, n_rows_pad, n_tab_rows, chunk, n_pass, n_src, stage):
    rpt = n_rows_pad // NS
    tpt = n_tab_rows // NS
    last = NS - 1
    cnt_last = n_rows - last * rpt
    n_tab = NC * n_pass
    QB = 20
    NQ = NBLK // QB
    KG = chunk // 16

    scratch = [
        pltpu.VMEM_SHARED((n_rows_pad, chunk), jnp.float32),
        pltpu.VMEM_SHARED((n_tab_rows, chunk), jnp.float32),
        pltpu.VMEM((QB, 128), jnp.int32),
        pltpu.VMEM((QB, 128), jnp.int32),
        pltpu.VMEM((128, chunk), jnp.float32),
        pltpu.VMEM((128, chunk), jnp.float32),
        pltpu.VMEM((128, chunk), jnp.float32),
        pltpu.VMEM((128, chunk), jnp.float32),
    ] + [pltpu.SemaphoreType.DMA] * 6

    def body(*refs):
        srcs = refs[:n_src]
        src_hbm, dst_hbm = refs[n_src], refs[n_src + 1]
        projs = refs[n_src + 2:n_src + n_tab + 2]
        zeros_hbm = refs[n_src + n_tab + 2]
        outs = refs[n_src + n_tab + 3:n_src + 2 * n_tab + 3]
        (accum, tabsh, src_v, dst_v, gbufA, gbufB, pbufA, pbufB,
         gsA, gsB, psA, psB, ssA, ssB) = refs[n_src + 2 * n_tab + 3:]
        c = lax.axis_index("c")
        s = lax.axis_index("s")

        def compute(gb, pb):
            def rows(r4, cc):
                for dr in range(4):
                    r = r4 * 4 + dr
                    for k in range(KG):
                        sl = pl.ds(k * 16, 16)
                        gb[r, sl] = jnp.maximum(gb[r, sl] + pb[r, sl], 0.0)
                return cc
            lax.fori_loop(0, 32, rows, 0)

        def run_pass(ci, p, proj, out_hbm):
            pltpu.sync_copy(zeros_hbm, accum.at[pl.ds(s * rpt, rpt)])
            stage(srcs, tabsh, ci, p, s, tpt)
            plsc.subcore_barrier()

            def issue(lrow, q, gb, pb, gsem, psem):
                e0 = s * EPT + (q * QB + lrow) * 128
                pltpu.async_copy(tabsh.at[src_v.at[lrow]], gb, gsem)
                pltpu.async_copy(proj.at[pl.ds(e0, 128)], pb, psem)

            def wait_gp(lrow, q, gb, pb, gsem, psem):
                e0 = s * EPT + (q * QB + lrow) * 128
                pltpu.make_async_copy(tabsh.at[src_v.at[lrow]], gb,
                                      gsem).wait()
                pltpu.make_async_copy(proj.at[pl.ds(e0, 128)], pb,
                                      psem).wait()

            def quarter(q, cc):
                @pl.when(q > 0)
                def _():
                    pltpu.make_async_copy(
                        gbufA, accum.at[dst_v.at[QB - 2]], ssA).wait()
                    pltpu.make_async_copy(
                        gbufB, accum.at[dst_v.at[QB - 1]], ssB).wait()
                r0 = s * NBLK + q * QB
                pltpu.sync_copy(src_hbm.at[pl.ds(r0, QB)], src_v)
                pltpu.sync_copy(dst_hbm.at[pl.ds(r0, QB)], dst_v)
                issue(0, q, gbufA, pbufA, gsA, psA)
                issue(1, q, gbufB, pbufB, gsB, psB)

                def pair(t, tc):
                    l0 = 2 * t
                    l1 = l0 + 1
                    wait_gp(l0, q, gbufA, pbufA, gsA, psA)
                    compute(gbufA, pbufA)
                    pltpu.async_copy(gbufA, accum.at[dst_v.at[l0]], ssA,
                                     add=True)
                    wait_gp(l1, q, gbufB, pbufB, gsB, psB)
                    compute(gbufB, pbufB)

                    @pl.when(t < QB // 2 - 1)
                    def _():
                        pltpu.make_async_copy(
                            gbufA, accum.at[dst_v.at[l0]], ssA).wait()
                        issue(l0 + 2, q, gbufA, pbufA, gsA, psA)
                    pltpu.async_copy(gbufB, accum.at[dst_v.at[l1]], ssB,
                                     add=True)

                    @pl.when(t < QB // 2 - 1)
                    def _():
                        pltpu.make_async_copy(
                            gbufB, accum.at[dst_v.at[l1]], ssB).wait()
                        issue(l1 + 2, q, gbufB, pbufB, gsB, psB)
                    return tc
                lax.fori_loop(0, QB // 2, pair, 0)
                return cc
            lax.fori_loop(0, NQ, quarter, 0)
            pltpu.make_async_copy(gbufA, accum.at[dst_v.at[QB - 2]],
                                  ssA).wait()
            pltpu.make_async_copy(gbufB, accum.at[dst_v.at[QB - 1]],
                                  ssB).wait()
            plsc.subcore_barrier()

            pl.when(s != last)(lambda: pltpu.sync_copy(
                accum.at[pl.ds(s * rpt, rpt)],
                out_hbm.at[pl.ds(s * rpt, rpt)]))
            pl.when(s == last)(lambda: pltpu.sync_copy(
                accum.at[pl.ds(last * rpt, cnt_last)],
                out_hbm.at[pl.ds(last * rpt, cnt_last)]))
            plsc.subcore_barrier()

        def run_passes(ci):
            for p in range(n_pass):
                t = n_pass * ci + p
                run_pass(ci, p, projs[t], outs[t])

        pl.when(c == 0)(lambda: run_passes(0))
        pl.when(c == 1)(lambda: run_passes(1))

    return functools.partial(
        pl.kernel, body,
        out_type=[jax.ShapeDtypeStruct((n_rows, chunk), jnp.float32)] * n_tab,
        mesh=_MESH,
        compiler_params=_SC_PARAMS,
        scratch_types=scratch,
    )()


def _stage1(srcs, tabsh, ci, p, s, tpt):
    col0 = (4 * ci + p) * 32
    pltpu.sync_copy(srcs[0].at[pl.ds(s * tpt, tpt), pl.ds(col0, 32)],
                    tabsh.at[pl.ds(s * tpt, tpt)])


def _stage2(srcs, tabsh, ci, p, s, tpt):
    sl = pl.ds(s * tpt, tpt)
    pltpu.sync_copy(srcs[2 * ci + p].at[sl], tabsh.at[sl])


_agg1 = _make_agg(N_FLAT, NF_PAD, N_FLAT, 32, 4, 1, _stage1)
_agg2 = _make_agg(N_TOT, NT_PAD, NT_PAD, 64, 2, 4, _stage2)


NFB = 156
NTAIL = 32


@functools.partial(
    pl.kernel,
    out_type=jax.ShapeDtypeStruct((N_FLAT, D), jnp.float32),
    mesh=_MESH,
    compiler_params=_SC_PARAMS,
    scratch_types=[
        pltpu.VMEM((128,), jnp.int32),
        pltpu.VMEM((32,), jnp.int32),
        pltpu.VMEM((128, D), jnp.float32),
        pltpu.VMEM((128, D), jnp.float32),
        pltpu.VMEM((4, D), jnp.float32),
        pltpu.SemaphoreType.DMA,
    ],
)
def _final_kernel(h1_hbm, h2_hbm, nid_hbm, aff_hbm, out_hbm,
                  nid_v, nid_t, b1v, b2v, aff_v, sem):
    c = lax.axis_index("c")
    s = lax.axis_index("s")
    w = s * NC + c
    pltpu.sync_copy(aff_hbm, aff_v)

    def compute_rows(nrows, h1buf, h2buf):
        def row(r, cc):
            for k in range(16):
                sl = pl.ds(k * 16, 16)
                v = (h1buf[r, sl] * aff_v[0, sl] + aff_v[1, sl]
                     + h2buf[r, sl] * aff_v[2, sl] + aff_v[3, sl])
                h1buf[r, sl] = jnp.maximum(v, 0.0)
            return cc
        lax.fori_loop(0, nrows, row, 0)

    def blk(t, carry):
        bi = w + 32 * t

        @pl.when(bi < NFB)
        def _():
            r0 = bi * 128
            pltpu.sync_copy(nid_hbm.at[pl.ds(r0, 128)], nid_v)
            pltpu.sync_copy(h1_hbm.at[pl.ds(r0, 128)], b1v)
            pltpu.async_copy(h2_hbm.at[nid_v], b2v, sem).wait()
            compute_rows(128, b1v, b2v)
            pltpu.sync_copy(b1v, out_hbm.at[pl.ds(r0, 128)])
        return carry
    lax.fori_loop(0, 5, blk, 0)

    @pl.when(w == 31)
    def _():
        r0 = NFB * 128
        pltpu.sync_copy(nid_hbm.at[pl.ds(r0, NTAIL)], nid_t)
        pltpu.sync_copy(h1_hbm.at[pl.ds(r0, NTAIL)], b1v.at[pl.ds(0, NTAIL)])
        pltpu.async_copy(h2_hbm.at[nid_t], b2v.at[pl.ds(0, NTAIL)],
                         sem).wait()
        compute_rows(NTAIL, b1v, b2v)
        pltpu.sync_copy(b1v.at[pl.ds(0, NTAIL)], out_hbm.at[pl.ds(r0, NTAIL)])


def _bn_affine(stats, gamma, beta, n):
    mu = stats[0] / n
    var = stats[1] / n - mu * mu
    sc = gamma * jax.lax.rsqrt(var + 1e-5)
    return sc, beta - mu * sc


def kernel(h_flat, intra_ei, ea_flat, valid, node_ids, N_total, edge_index,
           edge_attr, eps_l, We_l, be_l, W1_l, b1_l, W2_l, b2_l, g_l, bt_l,
           eps_g, We_g, be_g, W1_g, b1_g, W2_g, b2_g, g_g, bt_g):
    f32 = jnp.float32
    pad_e = E_PAD - E

    src1 = jnp.pad(intra_ei[0], (0, pad_e)).reshape(E_PAD // 128, 128)
    dst1 = jnp.pad(intra_ei[1], (0, pad_e),
                   constant_values=N_FLAT).reshape(E_PAD // 128, 128)
    src2 = jnp.pad(edge_index[0], (0, pad_e)).reshape(E_PAD // 128, 128)
    dst2 = jnp.pad(edge_index[1], (0, pad_e),
                   constant_values=N_TOT).reshape(E_PAD // 128, 128)
    nid_pad = jnp.pad(node_ids, (0, NF_PAD - N_FLAT), constant_values=N_TOT)
    h_pad = jnp.pad(h_flat, ((0, NF_PAD - N_FLAT), (0, 0)))

    ea1 = jnp.pad(ea_flat, ((0, pad_e), (0, 0)))
    ea2 = jnp.pad(edge_attr, ((0, pad_e), (0, 0)))
    proj1 = _edge_proj(ea1, We_l, be_l, 32)
    proj2 = _edge_proj(ea2, We_g, be_g, 64)

    zeros1 = jnp.zeros((NF_PAD // NS, 32), f32)
    zeros2 = jnp.zeros((NT_PAD // NS, 64), f32)
    zeros2w = jnp.zeros((NT_PAD // NS, 128), f32)
    zcnt = jnp.zeros((NT_PAD // NS, 16), f32)
    ones_c = jnp.ones((128, 16), f32)

    xt = _xsum_kernel(h_pad, nid_pad, zeros2w, zcnt, ones_c)

    agg1 = _agg1(h_flat, src1, dst1, *proj1, zeros1)
    two_col = [pl.BlockSpec((1000, 128), lambda i: (i, 0)),
               pl.BlockSpec((1000, 128), lambda i: (i, 1))]
    h1, stats1 = _mlp(eps_l, [h_flat, h_flat], two_col, agg1,
                      W1_l, b1_l, W2_l, b2_l, N_FLAT)

    agg2 = _agg2(*xt, src2, dst2, *proj2, zeros2)
    x_specs = [pl.BlockSpec((1000, 64), lambda i: (i, 0))] * 4
    h2, stats2 = _mlp(eps_g, list(xt), x_specs, agg2,
                      W1_g, b1_g, W2_g, b2_g, N_TOT)

    s_l, t_l = _bn_affine(stats1, g_l, bt_l, float(N_FLAT))
    s_g, t_g = _bn_affine(stats2, g_g, bt_g, float(N_TOT))
    aff = jnp.stack([s_l, t_l, s_g, t_g])

    return _final_kernel(h1, h2, node_ids, aff)
